# Initial kernel scaffold; baseline (speedup 1.0000x reference)
import jax
import jax.numpy as jnp
from jax.experimental import pallas as pl


def kernel(x):
    raise NotImplementedError("write your pallas kernel here")


# speedup vs baseline: 9.5842x; 9.5842x over previous
import functools

import jax
import jax.numpy as jnp
from jax.experimental import pallas as pl
from jax.experimental.pallas import tpu as pltpu

N = 4096
C = 128
K = 32
DILATION = 2
QBLK = 256


def _knn_kernel(x_q_ref, x_k_ref, out_ref):
    xq = x_q_ref[0]
    xk = x_k_ref[0]
    sq_q = jnp.sum(xq * xq, axis=-1, keepdims=True)
    sq_k = jnp.sum(xk * xk, axis=-1, keepdims=True).T
    inner = jax.lax.dot_general(
        xq, xk, (((1,), (1,)), ((), ())),
        preferred_element_type=jnp.float32,
        precision=jax.lax.Precision.DEFAULT)
    dist = sq_q - 2.0 * inner + sq_k

    iota = jax.lax.broadcasted_iota(jnp.int32, dist.shape, 1)
    big = jnp.float32(jnp.inf)
    cols = []
    for t in range(K - 1):
        m = jnp.min(dist, axis=1, keepdims=True)
        if t % 2 == 0:
            am = jnp.min(jnp.where(dist == m, iota, N), axis=1, keepdims=True)
            cols.append(am)
            dist = jnp.where(iota == am, big, dist)
        else:
            dist = jnp.where(dist == m, big, dist)
    out_ref[0] = jnp.concatenate(cols, axis=1)


@jax.jit
def kernel(x):
    b, n, c = x.shape
    grid = (b, n // QBLK)
    return pl.pallas_call(
        _knn_kernel,
        grid=grid,
        in_specs=[
            pl.BlockSpec((1, QBLK, C), lambda b, i: (b, i, 0)),
            pl.BlockSpec((1, N, C), lambda b, i: (b, 0, 0)),
        ],
        out_specs=pl.BlockSpec((1, QBLK, K // DILATION), lambda b, i: (b, i, 0)),
        out_shape=jax.ShapeDtypeStruct((b, n, K // DILATION), jnp.int32),
    )(x, x)
